# Initial kernel scaffold; baseline (speedup 1.0000x reference)
import jax
import jax.numpy as jnp
from jax.experimental import pallas as pl


def kernel(user_emb, item_emb, edge_index):
    raise NotImplementedError("write your pallas kernel here")


# speedup vs baseline: 9.1851x; 9.1851x over previous
import functools

import jax
import jax.numpy as jnp
from jax import lax
from jax.experimental import pallas as pl
from jax.experimental.pallas import tpu as pltpu
from jax.experimental.pallas import tpu_sc as plsc

N_USERS = 4000
N_ITEMS = 6000
N = N_USERS + N_ITEMS
D = 128
E = 320000
N_LAYERS = 3

NC = 2
NS = 16
NW = NC * NS
CHUNK = 128
CHUNKS_PW = 80
E_PAD = NW * CHUNK * CHUNKS_PW
N_PAD = 10240
RPT = N_PAD // NS


def _sc_mesh():
    return plsc.VectorSubcoreMesh(core_axis_name="c", subcore_axis_name="s")


def _deg_body(src_hbm, dst_hbm, out_hbm, dsrc_sh, ddst_sh, sidx, didx, ones_v):
    c = lax.axis_index("c")
    s = lax.axis_index("s")
    wid = c * NS + s

    def _zfill(i, _):
        ones_v[pl.ds(i * 16, 16)] = jnp.zeros((16,), jnp.float32)
        return 0
    lax.fori_loop(0, CHUNK // 16, _zfill, 0)
    for k in range(RPT // CHUNK):
        pltpu.sync_copy(ones_v, dsrc_sh.at[pl.ds(s * RPT + k * CHUNK, CHUNK)])
        pltpu.sync_copy(ones_v, ddst_sh.at[pl.ds(s * RPT + k * CHUNK, CHUNK)])

    def _fill(i, _):
        ones_v[pl.ds(i * 16, 16)] = jnp.ones((16,), jnp.float32)
        return 0
    lax.fori_loop(0, CHUNK // 16, _fill, 0)
    plsc.subcore_barrier()

    base = wid * (CHUNK * CHUNKS_PW)

    def _chunk(j, _):
        o = base + j * CHUNK
        pltpu.sync_copy(src_hbm.at[pl.ds(o, CHUNK)], sidx)
        pltpu.sync_copy(dst_hbm.at[pl.ds(o, CHUNK)], didx)
        pltpu.sync_copy(ones_v, dsrc_sh.at[sidx], add=True)
        pltpu.sync_copy(ones_v, ddst_sh.at[didx], add=True)
        return 0
    lax.fori_loop(0, CHUNKS_PW, _chunk, 0)
    plsc.subcore_barrier()

    pltpu.sync_copy(dsrc_sh.at[pl.ds(s * RPT, RPT)],
                    out_hbm.at[pl.ds((c * 2 + 0) * N_PAD + s * RPT, RPT)])
    pltpu.sync_copy(ddst_sh.at[pl.ds(s * RPT, RPT)],
                    out_hbm.at[pl.ds((c * 2 + 1) * N_PAD + s * RPT, RPT)])


@functools.lru_cache(maxsize=None)
def _deg_call():
    return pl.kernel(
        _deg_body,
        out_type=jax.ShapeDtypeStruct((4 * N_PAD,), jnp.float32),
        mesh=_sc_mesh(),
        scratch_types=[
            pltpu.VMEM_SHARED((N_PAD,), jnp.float32),
            pltpu.VMEM_SHARED((N_PAD,), jnp.float32),
            pltpu.VMEM((CHUNK,), jnp.int32),
            pltpu.VMEM((CHUNK,), jnp.int32),
            pltpu.VMEM((CHUNK,), jnp.float32),
        ],
    )


def _layer_body(u_hbm, src_hbm, dst_hbm, out_hbm, zsh, rows, sidx, didx, gsem):
    c = lax.axis_index("c")
    s = lax.axis_index("s")
    wid = c * NS + s

    def _zrow(i, _):
        for cc in range(D // 16):
            rows[i, pl.ds(cc * 16, 16)] = jnp.zeros((16,), jnp.float32)
        return 0
    lax.fori_loop(0, CHUNK, _zrow, 0)
    for k in range(RPT // CHUNK):
        pltpu.sync_copy(rows, zsh.at[pl.ds(s * RPT + k * CHUNK, CHUNK)])
    plsc.subcore_barrier()

    base = wid * (CHUNK * CHUNKS_PW)

    def _chunk(j, _):
        o = base + j * CHUNK
        pltpu.sync_copy(src_hbm.at[pl.ds(o, CHUNK)], sidx)
        pltpu.sync_copy(dst_hbm.at[pl.ds(o, CHUNK)], didx)
        pltpu.async_copy(u_hbm.at[sidx], rows, gsem).wait()
        pltpu.sync_copy(rows, zsh.at[didx], add=True)
        return 0
    lax.fori_loop(0, CHUNKS_PW, _chunk, 0)
    plsc.subcore_barrier()

    pltpu.sync_copy(zsh.at[pl.ds(s * RPT, RPT)],
                    out_hbm.at[pl.ds(c * N_PAD + s * RPT, RPT)])


@functools.lru_cache(maxsize=None)
def _layer_call():
    return pl.kernel(
        _layer_body,
        out_type=jax.ShapeDtypeStruct((2 * N_PAD, D), jnp.float32),
        mesh=_sc_mesh(),
        scratch_types=[
            pltpu.VMEM_SHARED((N_PAD, D), jnp.float32),
            pltpu.VMEM((CHUNK, D), jnp.float32),
            pltpu.VMEM((CHUNK,), jnp.int32),
            pltpu.VMEM((CHUNK,), jnp.int32),
            pltpu.SemaphoreType.DMA,
        ],
    )


_ROWS_BLK = 512


def _prep_body(dp_ref, x0_ref, u0_ref, su_ref, dd_ref):
    dp = dp_ref[...]
    degs = dp[0, 0] + dp[1, 0]
    degd = dp[0, 1] + dp[1, 1]
    dis = jnp.where(degs > 0, lax.rsqrt(jnp.maximum(degs, 1.0)), 0.0)
    did = jnp.where(degd > 0, lax.rsqrt(jnp.maximum(degd, 1.0)), 0.0)
    u0_ref[...] = x0_ref[...] * dis
    su_ref[...] = dis * did
    dd_ref[...] = did


@functools.lru_cache(maxsize=None)
def _prep_call():
    nblk = N_PAD // _ROWS_BLK
    return pl.pallas_call(
        _prep_body,
        grid=(nblk,),
        in_specs=[
            pl.BlockSpec((2, 2, _ROWS_BLK, 1), lambda i: (0, 0, i, 0)),
            pl.BlockSpec((_ROWS_BLK, D), lambda i: (i, 0)),
        ],
        out_specs=[
            pl.BlockSpec((_ROWS_BLK, D), lambda i: (i, 0)),
            pl.BlockSpec((_ROWS_BLK, 1), lambda i: (i, 0)),
            pl.BlockSpec((_ROWS_BLK, 1), lambda i: (i, 0)),
        ],
        out_shape=[
            jax.ShapeDtypeStruct((N_PAD, D), jnp.float32),
            jax.ShapeDtypeStruct((N_PAD, 1), jnp.float32),
            jax.ShapeDtypeStruct((N_PAD, 1), jnp.float32),
        ],
    )


def _merge_body(zp_ref, su_ref, dd_ref, acc_ref, unext_ref, accout_ref):
    z = zp_ref[0] + zp_ref[1]
    unext_ref[...] = z * su_ref[...]
    accout_ref[...] = acc_ref[...] + z * dd_ref[...]


def _final_body(zp_ref, dd_ref, acc_ref, out_ref):
    z = zp_ref[0] + zp_ref[1]
    out_ref[...] = (acc_ref[...] + z * dd_ref[...]) * 0.25


@functools.lru_cache(maxsize=None)
def _merge_call():
    nblk = N_PAD // _ROWS_BLK
    return pl.pallas_call(
        _merge_body,
        grid=(nblk,),
        in_specs=[
            pl.BlockSpec((2, _ROWS_BLK, D), lambda i: (0, i, 0)),
            pl.BlockSpec((_ROWS_BLK, 1), lambda i: (i, 0)),
            pl.BlockSpec((_ROWS_BLK, 1), lambda i: (i, 0)),
            pl.BlockSpec((_ROWS_BLK, D), lambda i: (i, 0)),
        ],
        out_specs=[
            pl.BlockSpec((_ROWS_BLK, D), lambda i: (i, 0)),
            pl.BlockSpec((_ROWS_BLK, D), lambda i: (i, 0)),
        ],
        out_shape=[
            jax.ShapeDtypeStruct((N_PAD, D), jnp.float32),
            jax.ShapeDtypeStruct((N_PAD, D), jnp.float32),
        ],
    )


@functools.lru_cache(maxsize=None)
def _final_call():
    nblk = N_PAD // _ROWS_BLK
    return pl.pallas_call(
        _final_body,
        grid=(nblk,),
        in_specs=[
            pl.BlockSpec((2, _ROWS_BLK, D), lambda i: (0, i, 0)),
            pl.BlockSpec((_ROWS_BLK, 1), lambda i: (i, 0)),
            pl.BlockSpec((_ROWS_BLK, D), lambda i: (i, 0)),
        ],
        out_specs=pl.BlockSpec((_ROWS_BLK, D), lambda i: (i, 0)),
        out_shape=jax.ShapeDtypeStruct((N_PAD, D), jnp.float32),
    )


def kernel(user_emb, item_emb, edge_index):
    x0 = jnp.concatenate([user_emb, item_emb], axis=0)
    x0p = jnp.pad(x0, ((0, N_PAD - N), (0, 0)))
    pad_idx = (N + (jnp.arange(E_PAD - E, dtype=jnp.int32) % (N_PAD - N))
               ).astype(jnp.int32)
    srcp = jnp.concatenate([edge_index[0], pad_idx])
    dstp = jnp.concatenate([edge_index[1], pad_idx])

    degpart = _deg_call()(srcp, dstp)
    dp = degpart.reshape(2, 2, N_PAD, 1)
    u, su, dd = _prep_call()(dp, x0p)

    acc = x0p
    out = None
    for l in range(N_LAYERS):
        zp = _layer_call()(u, srcp, dstp).reshape(2, N_PAD, D)
        if l < N_LAYERS - 1:
            u, acc = _merge_call()(zp, su, dd, acc)
        else:
            out = _final_call()(zp, dd, acc)
    return out[:N_USERS], out[N_USERS:N]


# speedup vs baseline: 16.0934x; 1.7521x over previous
import functools

import jax
import jax.numpy as jnp
from jax import lax
from jax.experimental import pallas as pl
from jax.experimental.pallas import tpu as pltpu
from jax.experimental.pallas import tpu_sc as plsc

N_USERS = 4000
N_ITEMS = 6000
N = N_USERS + N_ITEMS
D = 128
E = 320000
N_LAYERS = 3

NC = 2
NS = 16
NW = NC * NS
CHUNK = 128
CHUNKS_PW = 80
E_PAD = NW * CHUNK * CHUNKS_PW
N_PAD = 10240
RPT = N_PAD // NS
NBUF = 2


def _sc_mesh():
    return plsc.VectorSubcoreMesh(core_axis_name="c", subcore_axis_name="s")


def _deg_body(edges_hbm, out_hbm, dsrc_sh, ddst_sh, ib0, ib1, ones_v,
              sa0, sa1, sb0, sb1):
    c = lax.axis_index("c")
    s = lax.axis_index("s")
    wid = c * NS + s
    ib = (ib0, ib1)
    sa = (sa0, sa1)
    sb = (sb0, sb1)

    def _zfill(i, _):
        ones_v[pl.ds(i * 16, 16)] = jnp.zeros((16,), jnp.float32)
        return 0
    lax.fori_loop(0, CHUNK // 16, _zfill, 0)
    for k in range(RPT // CHUNK):
        pltpu.sync_copy(ones_v, dsrc_sh.at[pl.ds(s * RPT + k * CHUNK, CHUNK)])
        pltpu.sync_copy(ones_v, ddst_sh.at[pl.ds(s * RPT + k * CHUNK, CHUNK)])

    def _fill(i, _):
        ones_v[pl.ds(i * 16, 16)] = jnp.ones((16,), jnp.float32)
        return 0
    lax.fori_loop(0, CHUNK // 16, _fill, 0)
    plsc.subcore_barrier()

    base = wid * (CHUNK * CHUNKS_PW)

    def _pair(t, _):
        for kk in range(2):
            j = 2 * t + kk

            @pl.when(t >= 1)
            def _():
                pltpu.make_async_copy(ones_v, dsrc_sh.at[ib[kk].at[0]],
                                      sa[kk]).wait()
                pltpu.make_async_copy(ones_v, ddst_sh.at[ib[kk].at[1]],
                                      sb[kk]).wait()
            pltpu.sync_copy(edges_hbm.at[:, pl.ds(base + j * CHUNK, CHUNK)],
                            ib[kk])
            pltpu.async_copy(ones_v, dsrc_sh.at[ib[kk].at[0]], sa[kk],
                             add=True)
            pltpu.async_copy(ones_v, ddst_sh.at[ib[kk].at[1]], sb[kk],
                             add=True)
        return 0
    lax.fori_loop(0, CHUNKS_PW // 2, _pair, 0)
    for kk in range(2):
        pltpu.make_async_copy(ones_v, dsrc_sh.at[ib[kk].at[0]], sa[kk]).wait()
        pltpu.make_async_copy(ones_v, ddst_sh.at[ib[kk].at[1]], sb[kk]).wait()
    plsc.subcore_barrier()

    pltpu.sync_copy(dsrc_sh.at[pl.ds(s * RPT, RPT)],
                    out_hbm.at[pl.ds((c * 2 + 0) * N_PAD + s * RPT, RPT)])
    pltpu.sync_copy(ddst_sh.at[pl.ds(s * RPT, RPT)],
                    out_hbm.at[pl.ds((c * 2 + 1) * N_PAD + s * RPT, RPT)])


@functools.lru_cache(maxsize=None)
def _deg_call():
    return pl.kernel(
        _deg_body,
        out_type=jax.ShapeDtypeStruct((4 * N_PAD,), jnp.float32),
        mesh=_sc_mesh(),
        scratch_types=[
            pltpu.VMEM_SHARED((N_PAD,), jnp.float32),
            pltpu.VMEM_SHARED((N_PAD,), jnp.float32),
            pltpu.VMEM((2, CHUNK), jnp.int32),
            pltpu.VMEM((2, CHUNK), jnp.int32),
            pltpu.VMEM((CHUNK,), jnp.float32),
            pltpu.SemaphoreType.DMA,
            pltpu.SemaphoreType.DMA,
            pltpu.SemaphoreType.DMA,
            pltpu.SemaphoreType.DMA,
        ],
    )


def _layer_body(u_hbm, edges_hbm, out_hbm, zsh,
                r0, r1, i0, i1, g0, g1, s0, s1):
    c = lax.axis_index("c")
    s = lax.axis_index("s")
    wid = c * NS + s
    rows = (r0, r1)
    ib = (i0, i1)
    gsem = (g0, g1)
    ssem = (s0, s1)

    def _zrow(i, _):
        for cc in range(D // 16):
            r0[i, pl.ds(cc * 16, 16)] = jnp.zeros((16,), jnp.float32)
        return 0
    lax.fori_loop(0, CHUNK, _zrow, 0)
    for k in range(RPT // CHUNK):
        pltpu.sync_copy(r0, zsh.at[pl.ds(s * RPT + k * CHUNK, CHUNK)])
    plsc.subcore_barrier()

    base = wid * (CHUNK * CHUNKS_PW)

    def _quad(t, _):
        for kk in range(NBUF):
            j_is_first = (kk == 0)
            km1 = (kk - 1) % NBUF

            @pl.when(t >= 1)
            def _():
                pltpu.make_async_copy(rows[kk], zsh.at[ib[kk].at[1]],
                                      ssem[kk]).wait()
            pltpu.sync_copy(
                edges_hbm.at[:, pl.ds(base + (NBUF * t + kk) * CHUNK, CHUNK)],
                ib[kk])

            def _emit_prev_scatter():
                pltpu.make_async_copy(u_hbm.at[ib[km1].at[0]], rows[km1],
                                      gsem[km1]).wait()
                pltpu.async_copy(rows[km1], zsh.at[ib[km1].at[1]], ssem[km1],
                                 add=True)
            if j_is_first:
                pl.when(t >= 1)(_emit_prev_scatter)
            else:
                _emit_prev_scatter()
            pltpu.async_copy(u_hbm.at[ib[kk].at[0]], rows[kk], gsem[kk])
        return 0
    lax.fori_loop(0, CHUNKS_PW // NBUF, _quad, 0)

    last = NBUF - 1
    pltpu.make_async_copy(u_hbm.at[ib[last].at[0]], rows[last],
                          gsem[last]).wait()
    pltpu.async_copy(rows[last], zsh.at[ib[last].at[1]], ssem[last], add=True)
    for kk in range(NBUF):
        pltpu.make_async_copy(rows[kk], zsh.at[ib[kk].at[1]], ssem[kk]).wait()
    plsc.subcore_barrier()

    pltpu.sync_copy(zsh.at[pl.ds(s * RPT, RPT)],
                    out_hbm.at[pl.ds(c * N_PAD + s * RPT, RPT)])


@functools.lru_cache(maxsize=None)
def _layer_call():
    return pl.kernel(
        _layer_body,
        out_type=jax.ShapeDtypeStruct((2 * N_PAD, D), jnp.float32),
        mesh=_sc_mesh(),
        scratch_types=(
            [pltpu.VMEM_SHARED((N_PAD, D), jnp.float32)]
            + [pltpu.VMEM((CHUNK, D), jnp.float32)] * NBUF
            + [pltpu.VMEM((2, CHUNK), jnp.int32)] * NBUF
            + [pltpu.SemaphoreType.DMA] * (2 * NBUF)
        ),
    )


_ROWS_BLK = 512


def _prep_body(dp_ref, x0_ref, u0_ref, scl_ref):
    dp = dp_ref[...]
    degs = dp[0, 0] + dp[1, 0]
    degd = dp[0, 1] + dp[1, 1]
    dis = jnp.where(degs > 0, lax.rsqrt(jnp.maximum(degs, 1.0)), 0.0)
    did = jnp.where(degd > 0, lax.rsqrt(jnp.maximum(degd, 1.0)), 0.0)
    u0_ref[...] = x0_ref[...] * dis
    scl_ref[...] = jnp.concatenate([dis * did, did], axis=1)


@functools.lru_cache(maxsize=None)
def _prep_call():
    nblk = N_PAD // _ROWS_BLK
    return pl.pallas_call(
        _prep_body,
        grid=(nblk,),
        in_specs=[
            pl.BlockSpec((2, 2, _ROWS_BLK, 1), lambda i: (0, 0, i, 0)),
            pl.BlockSpec((_ROWS_BLK, D), lambda i: (i, 0)),
        ],
        out_specs=[
            pl.BlockSpec((_ROWS_BLK, D), lambda i: (i, 0)),
            pl.BlockSpec((_ROWS_BLK, 2), lambda i: (i, 0)),
        ],
        out_shape=[
            jax.ShapeDtypeStruct((N_PAD, D), jnp.float32),
            jax.ShapeDtypeStruct((N_PAD, 2), jnp.float32),
        ],
    )


def _merge_body(zp_ref, scl_ref, acc_ref, unext_ref, accout_ref):
    z = zp_ref[0] + zp_ref[1]
    unext_ref[...] = z * scl_ref[:, 0:1]
    accout_ref[...] = acc_ref[...] + z * scl_ref[:, 1:2]


def _final_body(zp_ref, scl_ref, acc_ref, out_ref):
    z = zp_ref[0] + zp_ref[1]
    out_ref[...] = (acc_ref[...] + z * scl_ref[:, 1:2]) * 0.25


@functools.lru_cache(maxsize=None)
def _merge_call():
    nblk = N_PAD // _ROWS_BLK
    return pl.pallas_call(
        _merge_body,
        grid=(nblk,),
        in_specs=[
            pl.BlockSpec((2, _ROWS_BLK, D), lambda i: (0, i, 0)),
            pl.BlockSpec((_ROWS_BLK, 2), lambda i: (i, 0)),
            pl.BlockSpec((_ROWS_BLK, D), lambda i: (i, 0)),
        ],
        out_specs=[
            pl.BlockSpec((_ROWS_BLK, D), lambda i: (i, 0)),
            pl.BlockSpec((_ROWS_BLK, D), lambda i: (i, 0)),
        ],
        out_shape=[
            jax.ShapeDtypeStruct((N_PAD, D), jnp.float32),
            jax.ShapeDtypeStruct((N_PAD, D), jnp.float32),
        ],
    )


@functools.lru_cache(maxsize=None)
def _final_call():
    nblk = N_PAD // _ROWS_BLK
    return pl.pallas_call(
        _final_body,
        grid=(nblk,),
        in_specs=[
            pl.BlockSpec((2, _ROWS_BLK, D), lambda i: (0, i, 0)),
            pl.BlockSpec((_ROWS_BLK, 2), lambda i: (i, 0)),
            pl.BlockSpec((_ROWS_BLK, D), lambda i: (i, 0)),
        ],
        out_specs=pl.BlockSpec((_ROWS_BLK, D), lambda i: (i, 0)),
        out_shape=jax.ShapeDtypeStruct((N_PAD, D), jnp.float32),
    )


def kernel(user_emb, item_emb, edge_index):
    x0 = jnp.concatenate([user_emb, item_emb], axis=0)
    x0p = jnp.pad(x0, ((0, N_PAD - N), (0, 0)))
    pad_idx = (N + (jnp.arange(E_PAD - E, dtype=jnp.int32) % (N_PAD - N))
               ).astype(jnp.int32)
    pad2 = jnp.stack([pad_idx, pad_idx])
    edges = jnp.concatenate([edge_index, pad2], axis=1)

    degpart = _deg_call()(edges)
    dp = degpart.reshape(2, 2, N_PAD, 1)
    u, scl = _prep_call()(dp, x0p)

    acc = x0p
    out = None
    for l in range(N_LAYERS):
        zp = _layer_call()(u, edges).reshape(2, N_PAD, D)
        if l < N_LAYERS - 1:
            u, acc = _merge_call()(zp, scl, acc)
        else:
            out = _final_call()(zp, scl, acc)
    return out[:N_USERS], out[N_USERS:N]


# speedup vs baseline: 19.1611x; 1.1906x over previous
import functools

import jax
import jax.numpy as jnp
from jax import lax
from jax.experimental import pallas as pl
from jax.experimental.pallas import tpu as pltpu
from jax.experimental.pallas import tpu_sc as plsc

N_USERS = 4000
N_ITEMS = 6000
N = N_USERS + N_ITEMS
D = 128
E = 320000
N_LAYERS = 3

NC = 2
NS = 16
NW = NC * NS
CHUNK = 128
CHUNKS_PW = 80
E_PAD = NW * CHUNK * CHUNKS_PW
N_PAD = 10240
RPT = N_PAD // NS
NBUF = 2


def _sc_mesh():
    return plsc.VectorSubcoreMesh(core_axis_name="c", subcore_axis_name="s")


def _deg_body(edges_hbm, out_hbm, dsrc_sh, ddst_sh, ib0, ib1, ib2, ib3,
              ones_v, sa0, sa1, sb0, sb1, is0, is1, is2, is3):
    c = lax.axis_index("c")
    s = lax.axis_index("s")
    wid = c * NS + s
    ib = (ib0, ib1, ib2, ib3)
    sa = (sa0, sa1)
    sb = (sb0, sb1)
    isem = (is0, is1, is2, is3)

    def _zfill(i, _):
        ones_v[pl.ds(i * 16, 16)] = jnp.zeros((16,), jnp.float32)
        return 0
    lax.fori_loop(0, CHUNK // 16, _zfill, 0)
    for k in range(RPT // CHUNK):
        pltpu.sync_copy(ones_v, dsrc_sh.at[pl.ds(s * RPT + k * CHUNK, CHUNK)])
        pltpu.sync_copy(ones_v, ddst_sh.at[pl.ds(s * RPT + k * CHUNK, CHUNK)])

    def _fill(i, _):
        ones_v[pl.ds(i * 16, 16)] = jnp.ones((16,), jnp.float32)
        return 0
    lax.fori_loop(0, CHUNK // 16, _fill, 0)
    plsc.subcore_barrier()

    base = wid * (CHUNK * CHUNKS_PW)

    pltpu.async_copy(edges_hbm.at[:, pl.ds(base, CHUNK)], ib[0], isem[0])
    pltpu.async_copy(edges_hbm.at[:, pl.ds(base + CHUNK, CHUNK)], ib[1],
                     isem[1])

    def _quad(t, _):
        for kk in range(4):
            j = 4 * t + kk
            ks = kk % 2

            def _wait_prev():
                pltpu.make_async_copy(ones_v, dsrc_sh.at[ib[kk].at[0]],
                                      sa[ks]).wait()
                pltpu.make_async_copy(ones_v, ddst_sh.at[ib[kk].at[1]],
                                      sb[ks]).wait()
            if kk < 2:
                pl.when(t >= 1)(_wait_prev)
            else:
                _wait_prev()
            pltpu.make_async_copy(
                edges_hbm.at[:, pl.ds(base + j * CHUNK, CHUNK)],
                ib[kk], isem[kk]).wait()
            pltpu.async_copy(ones_v, dsrc_sh.at[ib[kk].at[0]], sa[ks],
                             add=True)
            pltpu.async_copy(ones_v, ddst_sh.at[ib[kk].at[1]], sb[ks],
                             add=True)

            def _prefetch():
                jn = j + 2
                kn = (kk + 2) % 4
                pltpu.async_copy(
                    edges_hbm.at[:, pl.ds(base + jn * CHUNK, CHUNK)],
                    ib[kn], isem[kn])
            if kk < 2:
                _prefetch()
            else:
                pl.when(t < CHUNKS_PW // 4 - 1)(_prefetch)
        return 0
    lax.fori_loop(0, CHUNKS_PW // 4, _quad, 0)
    for kk in range(2):
        pltpu.make_async_copy(ones_v, dsrc_sh.at[ib[kk].at[0]], sa[kk]).wait()
        pltpu.make_async_copy(ones_v, ddst_sh.at[ib[kk].at[1]], sb[kk]).wait()
    plsc.subcore_barrier()

    pltpu.sync_copy(dsrc_sh.at[pl.ds(s * RPT, RPT)],
                    out_hbm.at[pl.ds((c * 2 + 0) * N_PAD + s * RPT, RPT)])
    pltpu.sync_copy(ddst_sh.at[pl.ds(s * RPT, RPT)],
                    out_hbm.at[pl.ds((c * 2 + 1) * N_PAD + s * RPT, RPT)])


@functools.lru_cache(maxsize=None)
def _deg_call():
    return pl.kernel(
        _deg_body,
        out_type=jax.ShapeDtypeStruct((4 * N_PAD,), jnp.float32),
        mesh=_sc_mesh(),
        scratch_types=[
            pltpu.VMEM_SHARED((N_PAD,), jnp.float32),
            pltpu.VMEM_SHARED((N_PAD,), jnp.float32),
            pltpu.VMEM((2, CHUNK), jnp.int32),
            pltpu.VMEM((2, CHUNK), jnp.int32),
            pltpu.VMEM((2, CHUNK), jnp.int32),
            pltpu.VMEM((2, CHUNK), jnp.int32),
            pltpu.VMEM((CHUNK,), jnp.float32),
        ] + [pltpu.SemaphoreType.DMA] * 8,
    )


def _layer_body(u_hbm, edges_hbm, out_hbm, zsh,
                r0, r1, i0, i1, i2, i3,
                g0, g1, s0, s1, is0, is1, is2, is3):
    c = lax.axis_index("c")
    s = lax.axis_index("s")
    wid = c * NS + s
    rows = (r0, r1)
    ib = (i0, i1, i2, i3)
    gsem = (g0, g1)
    ssem = (s0, s1)
    isem = (is0, is1, is2, is3)

    def _zrow(i, _):
        for cc in range(D // 16):
            r0[i, pl.ds(cc * 16, 16)] = jnp.zeros((16,), jnp.float32)
        return 0
    lax.fori_loop(0, CHUNK, _zrow, 0)
    for k in range(RPT // CHUNK):
        pltpu.sync_copy(r0, zsh.at[pl.ds(s * RPT + k * CHUNK, CHUNK)])
    plsc.subcore_barrier()

    base = wid * (CHUNK * CHUNKS_PW)

    pltpu.async_copy(edges_hbm.at[:, pl.ds(base, CHUNK)], ib[0], isem[0])
    pltpu.async_copy(edges_hbm.at[:, pl.ds(base + CHUNK, CHUNK)], ib[1],
                     isem[1])

    def _quad(t, _):
        for kk in range(4):
            j = 4 * t + kk
            kr = kk % 2
            kp = (kk - 1) % 4

            def _wait_rows_free():
                pltpu.make_async_copy(rows[kr], zsh.at[ib[kk].at[1]],
                                      ssem[kr]).wait()
            if kk < 2:
                pl.when(t >= 1)(_wait_rows_free)
            else:
                _wait_rows_free()
            pltpu.make_async_copy(
                edges_hbm.at[:, pl.ds(base + j * CHUNK, CHUNK)],
                ib[kk], isem[kk]).wait()
            pltpu.async_copy(u_hbm.at[ib[kk].at[0]], rows[kr], gsem[kr])

            def _prefetch():
                jn = j + 2
                kn = (kk + 2) % 4
                pltpu.async_copy(
                    edges_hbm.at[:, pl.ds(base + jn * CHUNK, CHUNK)],
                    ib[kn], isem[kn])
            if kk < 2:
                _prefetch()
            else:
                pl.when(t < CHUNKS_PW // 4 - 1)(_prefetch)

            def _emit_prev_scatter():
                pltpu.make_async_copy(u_hbm.at[ib[kp].at[0]], rows[1 - kr],
                                      gsem[1 - kr]).wait()
                pltpu.async_copy(rows[1 - kr], zsh.at[ib[kp].at[1]],
                                 ssem[1 - kr], add=True)
            if kk == 0:
                pl.when(t >= 1)(_emit_prev_scatter)
            else:
                _emit_prev_scatter()
        return 0
    lax.fori_loop(0, CHUNKS_PW // 4, _quad, 0)

    pltpu.make_async_copy(u_hbm.at[ib[3].at[0]], rows[1], gsem[1]).wait()
    pltpu.async_copy(rows[1], zsh.at[ib[3].at[1]], ssem[1], add=True)
    pltpu.make_async_copy(rows[0], zsh.at[ib[2].at[1]], ssem[0]).wait()
    pltpu.make_async_copy(rows[1], zsh.at[ib[3].at[1]], ssem[1]).wait()
    plsc.subcore_barrier()

    pltpu.sync_copy(zsh.at[pl.ds(s * RPT, RPT)],
                    out_hbm.at[pl.ds(c * N_PAD + s * RPT, RPT)])


@functools.lru_cache(maxsize=None)
def _layer_call():
    return pl.kernel(
        _layer_body,
        out_type=jax.ShapeDtypeStruct((2 * N_PAD, D), jnp.float32),
        mesh=_sc_mesh(),
        scratch_types=(
            [pltpu.VMEM_SHARED((N_PAD, D), jnp.float32)]
            + [pltpu.VMEM((CHUNK, D), jnp.float32)] * 2
            + [pltpu.VMEM((2, CHUNK), jnp.int32)] * 4
            + [pltpu.SemaphoreType.DMA] * 8
        ),
    )


_ROWS_BLK = 512


def _prep_body(dp_ref, x0_ref, u0_ref, scl_ref):
    dp = dp_ref[...]
    degs = dp[0, 0] + dp[1, 0]
    degd = dp[0, 1] + dp[1, 1]
    dis = jnp.where(degs > 0, lax.rsqrt(jnp.maximum(degs, 1.0)), 0.0)
    did = jnp.where(degd > 0, lax.rsqrt(jnp.maximum(degd, 1.0)), 0.0)
    u0_ref[...] = x0_ref[...] * dis
    scl_ref[...] = jnp.concatenate([dis * did, did], axis=1)


@functools.lru_cache(maxsize=None)
def _prep_call():
    nblk = N_PAD // _ROWS_BLK
    return pl.pallas_call(
        _prep_body,
        grid=(nblk,),
        in_specs=[
            pl.BlockSpec((2, 2, _ROWS_BLK, 1), lambda i: (0, 0, i, 0)),
            pl.BlockSpec((_ROWS_BLK, D), lambda i: (i, 0)),
        ],
        out_specs=[
            pl.BlockSpec((_ROWS_BLK, D), lambda i: (i, 0)),
            pl.BlockSpec((_ROWS_BLK, 2), lambda i: (i, 0)),
        ],
        out_shape=[
            jax.ShapeDtypeStruct((N_PAD, D), jnp.float32),
            jax.ShapeDtypeStruct((N_PAD, 2), jnp.float32),
        ],
    )


def _merge_body(zp_ref, scl_ref, acc_ref, unext_ref, accout_ref):
    z = zp_ref[0] + zp_ref[1]
    unext_ref[...] = z * scl_ref[:, 0:1]
    accout_ref[...] = acc_ref[...] + z * scl_ref[:, 1:2]


def _final_body(zp_ref, scl_ref, acc_ref, out_ref):
    z = zp_ref[0] + zp_ref[1]
    out_ref[...] = (acc_ref[...] + z * scl_ref[:, 1:2]) * 0.25


@functools.lru_cache(maxsize=None)
def _merge_call():
    nblk = N_PAD // _ROWS_BLK
    return pl.pallas_call(
        _merge_body,
        grid=(nblk,),
        in_specs=[
            pl.BlockSpec((2, _ROWS_BLK, D), lambda i: (0, i, 0)),
            pl.BlockSpec((_ROWS_BLK, 2), lambda i: (i, 0)),
            pl.BlockSpec((_ROWS_BLK, D), lambda i: (i, 0)),
        ],
        out_specs=[
            pl.BlockSpec((_ROWS_BLK, D), lambda i: (i, 0)),
            pl.BlockSpec((_ROWS_BLK, D), lambda i: (i, 0)),
        ],
        out_shape=[
            jax.ShapeDtypeStruct((N_PAD, D), jnp.float32),
            jax.ShapeDtypeStruct((N_PAD, D), jnp.float32),
        ],
    )


@functools.lru_cache(maxsize=None)
def _final_call():
    nblk = N_PAD // _ROWS_BLK
    return pl.pallas_call(
        _final_body,
        grid=(nblk,),
        in_specs=[
            pl.BlockSpec((2, _ROWS_BLK, D), lambda i: (0, i, 0)),
            pl.BlockSpec((_ROWS_BLK, 2), lambda i: (i, 0)),
            pl.BlockSpec((_ROWS_BLK, D), lambda i: (i, 0)),
        ],
        out_specs=pl.BlockSpec((_ROWS_BLK, D), lambda i: (i, 0)),
        out_shape=jax.ShapeDtypeStruct((N_PAD, D), jnp.float32),
    )


def kernel(user_emb, item_emb, edge_index):
    x0 = jnp.concatenate([user_emb, item_emb], axis=0)
    x0p = jnp.pad(x0, ((0, N_PAD - N), (0, 0)))
    pad_idx = (N + (jnp.arange(E_PAD - E, dtype=jnp.int32) % (N_PAD - N))
               ).astype(jnp.int32)
    pad2 = jnp.stack([pad_idx, pad_idx])
    edges = jnp.concatenate([edge_index, pad2], axis=1)

    degpart = _deg_call()(edges)
    dp = degpart.reshape(2, 2, N_PAD, 1)
    u, scl = _prep_call()(dp, x0p)

    acc = x0p
    out = None
    for l in range(N_LAYERS):
        zp = _layer_call()(u, edges).reshape(2, N_PAD, D)
        if l < N_LAYERS - 1:
            u, acc = _merge_call()(zp, scl, acc)
        else:
            out = _final_call()(zp, scl, acc)
    return out[:N_USERS], out[N_USERS:N]


# speedup vs baseline: 19.1754x; 1.0007x over previous
import functools

import jax
import jax.numpy as jnp
from jax import lax
from jax.experimental import pallas as pl
from jax.experimental.pallas import tpu as pltpu
from jax.experimental.pallas import tpu_sc as plsc

N_USERS = 4000
N_ITEMS = 6000
N = N_USERS + N_ITEMS
D = 128
E = 320000
N_LAYERS = 3

NC = 2
NS = 16
NW = NC * NS
CHUNK = 128
CHUNKS_PW = 80
E_PAD = NW * CHUNK * CHUNKS_PW
N_PAD = 10240
RPT = N_PAD // NS
NBUF = 2


def _sc_mesh():
    return plsc.VectorSubcoreMesh(core_axis_name="c", subcore_axis_name="s")


def _deg_body(edges_hbm, out_hbm, dsrc_sh, ddst_sh, ibuf, ones_v,
              sa0, sa1, sb0, sb1, isem):
    c = lax.axis_index("c")
    s = lax.axis_index("s")
    wid = c * NS + s
    sa = (sa0, sa1)
    sb = (sb0, sb1)

    idesc = pltpu.async_copy(edges_hbm.at[:, wid], ibuf, isem)

    def _zfill(i, _):
        ones_v[pl.ds(i * 16, 16)] = jnp.zeros((16,), jnp.float32)
        return 0
    lax.fori_loop(0, CHUNK // 16, _zfill, 0)
    for off in range(0, RPT, CHUNK):
        w = min(CHUNK, RPT - off)
        pltpu.sync_copy(ones_v.at[pl.ds(0, w)],
                        dsrc_sh.at[pl.ds(s * RPT + off, w)])
        pltpu.sync_copy(ones_v.at[pl.ds(0, w)],
                        ddst_sh.at[pl.ds(s * RPT + off, w)])

    def _fill(i, _):
        ones_v[pl.ds(i * 16, 16)] = jnp.ones((16,), jnp.float32)
        return 0
    lax.fori_loop(0, CHUNK // 16, _fill, 0)
    idesc.wait()
    plsc.subcore_barrier()

    def _pair(t, _):
        for kk in range(2):
            j = 2 * t + kk

            def _wait_prev():
                pltpu.make_async_copy(ones_v, dsrc_sh.at[ibuf.at[0, j]],
                                      sa[kk]).wait()
                pltpu.make_async_copy(ones_v, ddst_sh.at[ibuf.at[1, j]],
                                      sb[kk]).wait()
            if kk == 0:
                pl.when(t >= 1)(_wait_prev)
            else:
                pl.when(t >= 1)(_wait_prev)
            pltpu.async_copy(ones_v, dsrc_sh.at[ibuf.at[0, j]], sa[kk],
                             add=True)
            pltpu.async_copy(ones_v, ddst_sh.at[ibuf.at[1, j]], sb[kk],
                             add=True)
        return 0
    lax.fori_loop(0, CHUNKS_PW // 2, _pair, 0)
    for kk in range(2):
        j = CHUNKS_PW - 2 + kk
        pltpu.make_async_copy(ones_v, dsrc_sh.at[ibuf.at[0, j]], sa[kk]).wait()
        pltpu.make_async_copy(ones_v, ddst_sh.at[ibuf.at[1, j]], sb[kk]).wait()
    plsc.subcore_barrier()

    pltpu.sync_copy(dsrc_sh.at[pl.ds(s * RPT, RPT)],
                    out_hbm.at[pl.ds((c * 2 + 0) * N_PAD + s * RPT, RPT)])
    pltpu.sync_copy(ddst_sh.at[pl.ds(s * RPT, RPT)],
                    out_hbm.at[pl.ds((c * 2 + 1) * N_PAD + s * RPT, RPT)])


@functools.lru_cache(maxsize=None)
def _deg_call():
    return pl.kernel(
        _deg_body,
        out_type=jax.ShapeDtypeStruct((4 * N_PAD,), jnp.float32),
        mesh=_sc_mesh(),
        scratch_types=[
            pltpu.VMEM_SHARED((N_PAD,), jnp.float32),
            pltpu.VMEM_SHARED((N_PAD,), jnp.float32),
            pltpu.VMEM((2, CHUNKS_PW, CHUNK), jnp.int32),
            pltpu.VMEM((CHUNK,), jnp.float32),
        ] + [pltpu.SemaphoreType.DMA] * 5,
    )


def _layer_body(u_hbm, edges_hbm, zeros_hbm, out_hbm, zsh,
                r0, r1, i0, i1, i2, i3,
                g0, g1, s0, s1, is0, is1, is2, is3, zsem):
    c = lax.axis_index("c")
    s = lax.axis_index("s")
    wid = c * NS + s
    rows = (r0, r1)
    ib = (i0, i1, i2, i3)
    gsem = (g0, g1)
    ssem = (s0, s1)
    isem = (is0, is1, is2, is3)

    zdesc = pltpu.async_copy(zeros_hbm.at[pl.ds(s * RPT, RPT)],
                             zsh.at[pl.ds(s * RPT, RPT)], zsem)
    base = wid * CHUNKS_PW

    pltpu.async_copy(edges_hbm.at[:, pl.ds(base, 1)], ib[0], isem[0])
    pltpu.async_copy(edges_hbm.at[:, pl.ds(base + 1, 1)], ib[1], isem[1])
    zdesc.wait()
    plsc.subcore_barrier()

    def _quad(t, _):
        for kk in range(4):
            j = 4 * t + kk
            kr = kk % 2
            kp = (kk - 1) % 4

            def _wait_rows_free():
                pltpu.make_async_copy(rows[kr], zsh.at[ib[kk].at[0, 0]],
                                      ssem[kr]).wait()
            if kk < 2:
                pl.when(t >= 1)(_wait_rows_free)
            else:
                _wait_rows_free()
            pltpu.make_async_copy(
                edges_hbm.at[:, pl.ds(base + j, 1)], ib[kk],
                isem[kk]).wait()
            pltpu.async_copy(u_hbm.at[ib[kk].at[0, 0]], rows[kr], gsem[kr])

            def _prefetch():
                jn = j + 2
                kn = (kk + 2) % 4
                pltpu.async_copy(edges_hbm.at[:, pl.ds(base + jn, 1)],
                                 ib[kn], isem[kn])
            if kk < 2:
                _prefetch()
            else:
                pl.when(t < CHUNKS_PW // 4 - 1)(_prefetch)

            def _emit_prev_scatter():
                pltpu.make_async_copy(u_hbm.at[ib[kp].at[0, 0]], rows[1 - kr],
                                      gsem[1 - kr]).wait()
                pltpu.async_copy(rows[1 - kr], zsh.at[ib[kp].at[1, 0]],
                                 ssem[1 - kr], add=True)
            if kk == 0:
                pl.when(t >= 1)(_emit_prev_scatter)
            else:
                _emit_prev_scatter()
        return 0
    lax.fori_loop(0, CHUNKS_PW // 4, _quad, 0)

    pltpu.make_async_copy(u_hbm.at[ib[3].at[0, 0]], rows[1], gsem[1]).wait()
    pltpu.async_copy(rows[1], zsh.at[ib[3].at[1, 0]], ssem[1], add=True)
    pltpu.make_async_copy(rows[0], zsh.at[ib[2].at[1, 0]], ssem[0]).wait()
    pltpu.make_async_copy(rows[1], zsh.at[ib[3].at[1, 0]], ssem[1]).wait()
    plsc.subcore_barrier()

    pltpu.sync_copy(zsh.at[pl.ds(s * RPT, RPT)],
                    out_hbm.at[pl.ds(c * N_PAD + s * RPT, RPT)])


@functools.lru_cache(maxsize=None)
def _layer_call():
    return pl.kernel(
        _layer_body,
        out_type=jax.ShapeDtypeStruct((2 * N_PAD, D), jnp.float32),
        mesh=_sc_mesh(),
        scratch_types=(
            [pltpu.VMEM_SHARED((N_PAD, D), jnp.float32)]
            + [pltpu.VMEM((CHUNK, D), jnp.float32)] * 2
            + [pltpu.VMEM((2, 1, CHUNK), jnp.int32)] * 4
            + [pltpu.SemaphoreType.DMA] * 9
        ),
    )


_ROWS_BLK = 512


def _prep_body(dp_ref, x0_ref, u0_ref, scl_ref):
    dp = dp_ref[...]
    degs = dp[0, 0] + dp[1, 0]
    degd = dp[0, 1] + dp[1, 1]
    dis = jnp.where(degs > 0, lax.rsqrt(jnp.maximum(degs, 1.0)), 0.0)
    did = jnp.where(degd > 0, lax.rsqrt(jnp.maximum(degd, 1.0)), 0.0)
    u0_ref[...] = x0_ref[...] * dis
    scl_ref[...] = jnp.concatenate([dis * did, did], axis=1)


@functools.lru_cache(maxsize=None)
def _prep_call():
    nblk = N_PAD // _ROWS_BLK
    return pl.pallas_call(
        _prep_body,
        grid=(nblk,),
        in_specs=[
            pl.BlockSpec((2, 2, _ROWS_BLK, 1), lambda i: (0, 0, i, 0)),
            pl.BlockSpec((_ROWS_BLK, D), lambda i: (i, 0)),
        ],
        out_specs=[
            pl.BlockSpec((_ROWS_BLK, D), lambda i: (i, 0)),
            pl.BlockSpec((_ROWS_BLK, 2), lambda i: (i, 0)),
        ],
        out_shape=[
            jax.ShapeDtypeStruct((N_PAD, D), jnp.float32),
            jax.ShapeDtypeStruct((N_PAD, 2), jnp.float32),
        ],
    )


def _merge_body(zp_ref, scl_ref, acc_ref, unext_ref, accout_ref):
    z = zp_ref[0] + zp_ref[1]
    unext_ref[...] = z * scl_ref[:, 0:1]
    accout_ref[...] = acc_ref[...] + z * scl_ref[:, 1:2]


def _final_body(zp_ref, scl_ref, acc_ref, out_ref):
    z = zp_ref[0] + zp_ref[1]
    out_ref[...] = (acc_ref[...] + z * scl_ref[:, 1:2]) * 0.25


@functools.lru_cache(maxsize=None)
def _merge_call():
    nblk = N_PAD // _ROWS_BLK
    return pl.pallas_call(
        _merge_body,
        grid=(nblk,),
        in_specs=[
            pl.BlockSpec((2, _ROWS_BLK, D), lambda i: (0, i, 0)),
            pl.BlockSpec((_ROWS_BLK, 2), lambda i: (i, 0)),
            pl.BlockSpec((_ROWS_BLK, D), lambda i: (i, 0)),
        ],
        out_specs=[
            pl.BlockSpec((_ROWS_BLK, D), lambda i: (i, 0)),
            pl.BlockSpec((_ROWS_BLK, D), lambda i: (i, 0)),
        ],
        out_shape=[
            jax.ShapeDtypeStruct((N_PAD, D), jnp.float32),
            jax.ShapeDtypeStruct((N_PAD, D), jnp.float32),
        ],
    )


@functools.lru_cache(maxsize=None)
def _final_call():
    nblk = N_PAD // _ROWS_BLK
    return pl.pallas_call(
        _final_body,
        grid=(nblk,),
        in_specs=[
            pl.BlockSpec((2, _ROWS_BLK, D), lambda i: (0, i, 0)),
            pl.BlockSpec((_ROWS_BLK, 2), lambda i: (i, 0)),
            pl.BlockSpec((_ROWS_BLK, D), lambda i: (i, 0)),
        ],
        out_specs=pl.BlockSpec((_ROWS_BLK, D), lambda i: (i, 0)),
        out_shape=jax.ShapeDtypeStruct((N_PAD, D), jnp.float32),
    )


def kernel(user_emb, item_emb, edge_index):
    x0 = jnp.concatenate([user_emb, item_emb], axis=0)
    x0p = jnp.pad(x0, ((0, N_PAD - N), (0, 0)))
    pad_idx = (N + (jnp.arange(E_PAD - E, dtype=jnp.int32) % (N_PAD - N))
               ).astype(jnp.int32)
    pad2 = jnp.stack([pad_idx, pad_idx])
    edges = jnp.concatenate([edge_index, pad2], axis=1)
    edges4 = edges.reshape(2, NW, CHUNKS_PW, CHUNK)
    edges3 = edges.reshape(2, NW * CHUNKS_PW, CHUNK)
    zeros = jnp.zeros((N_PAD, D), jnp.float32)

    degpart = _deg_call()(edges4)
    dp = degpart.reshape(2, 2, N_PAD, 1)
    u, scl = _prep_call()(dp, x0p)

    acc = x0p
    out = None
    for l in range(N_LAYERS):
        zp = _layer_call()(u, edges3, zeros).reshape(2, N_PAD, D)
        if l < N_LAYERS - 1:
            u, acc = _merge_call()(zp, scl, acc)
        else:
            out = _final_call()(zp, scl, acc)
    return out[:N_USERS], out[N_USERS:N]


# speedup vs baseline: 20.1957x; 1.0532x over previous
import functools

import jax
import jax.numpy as jnp
from jax import lax
from jax.experimental import pallas as pl
from jax.experimental.pallas import tpu as pltpu
from jax.experimental.pallas import tpu_sc as plsc

N_USERS = 4000
N_ITEMS = 6000
N = N_USERS + N_ITEMS
D = 128
E = 320000
N_LAYERS = 3

NC = 2
NS = 16
NW = NC * NS
CHUNK = 128
CHUNKS_PW = 80
E_PAD = NW * CHUNK * CHUNKS_PW
N_PAD = 10240
RPT = N_PAD // NS
NBUF = 2


def _sc_mesh():
    return plsc.VectorSubcoreMesh(core_axis_name="c", subcore_axis_name="s")


def _deg_body(edges_hbm, out_hbm, dsrc_sh, ddst_sh, ibuf, ones_v,
              sa0, sa1, sa2, sa3, sb0, sb1, sb2, sb3, isem):
    c = lax.axis_index("c")
    s = lax.axis_index("s")
    wid = c * NS + s
    sa = (sa0, sa1, sa2, sa3)
    sb = (sb0, sb1, sb2, sb3)

    idesc = pltpu.async_copy(edges_hbm.at[:, wid], ibuf, isem)

    def _zfill(i, _):
        ones_v[pl.ds(i * 16, 16)] = jnp.zeros((16,), jnp.float32)
        return 0
    lax.fori_loop(0, CHUNK // 16, _zfill, 0)
    for off in range(0, RPT, CHUNK):
        w = min(CHUNK, RPT - off)
        pltpu.sync_copy(ones_v.at[pl.ds(0, w)],
                        dsrc_sh.at[pl.ds(s * RPT + off, w)])
        pltpu.sync_copy(ones_v.at[pl.ds(0, w)],
                        ddst_sh.at[pl.ds(s * RPT + off, w)])

    def _fill(i, _):
        ones_v[pl.ds(i * 16, 16)] = jnp.ones((16,), jnp.float32)
        return 0
    lax.fori_loop(0, CHUNK // 16, _fill, 0)
    idesc.wait()
    plsc.subcore_barrier()

    def _quad(t, _):
        for kk in range(4):
            j = 4 * t + kk

            def _wait_prev():
                pltpu.make_async_copy(ones_v, dsrc_sh.at[ibuf.at[0, j]],
                                      sa[kk]).wait()
                pltpu.make_async_copy(ones_v, ddst_sh.at[ibuf.at[1, j]],
                                      sb[kk]).wait()
            pl.when(t >= 1)(_wait_prev)
            pltpu.async_copy(ones_v, dsrc_sh.at[ibuf.at[0, j]], sa[kk],
                             add=True)
            pltpu.async_copy(ones_v, ddst_sh.at[ibuf.at[1, j]], sb[kk],
                             add=True)
        return 0
    lax.fori_loop(0, CHUNKS_PW // 4, _quad, 0)
    for kk in range(4):
        j = CHUNKS_PW - 4 + kk
        pltpu.make_async_copy(ones_v, dsrc_sh.at[ibuf.at[0, j]], sa[kk]).wait()
        pltpu.make_async_copy(ones_v, ddst_sh.at[ibuf.at[1, j]], sb[kk]).wait()
    plsc.subcore_barrier()

    pltpu.sync_copy(dsrc_sh.at[pl.ds(s * RPT, RPT)],
                    out_hbm.at[pl.ds((c * 2 + 0) * N_PAD + s * RPT, RPT)])
    pltpu.sync_copy(ddst_sh.at[pl.ds(s * RPT, RPT)],
                    out_hbm.at[pl.ds((c * 2 + 1) * N_PAD + s * RPT, RPT)])


@functools.lru_cache(maxsize=None)
def _deg_call():
    return pl.kernel(
        _deg_body,
        out_type=jax.ShapeDtypeStruct((4 * N_PAD,), jnp.float32),
        mesh=_sc_mesh(),
        scratch_types=[
            pltpu.VMEM_SHARED((N_PAD,), jnp.float32),
            pltpu.VMEM_SHARED((N_PAD,), jnp.float32),
            pltpu.VMEM((2, CHUNKS_PW, CHUNK), jnp.int32),
            pltpu.VMEM((CHUNK,), jnp.float32),
        ] + [pltpu.SemaphoreType.DMA] * 9,
    )


def _layer_body(u_hbm, edges_hbm, zeros_hbm, out_hbm, zsh,
                r0, r1, i0, i1, i2, i3,
                g0, g1, s0, s1, is0, is1, is2, is3, zsem):
    c = lax.axis_index("c")
    s = lax.axis_index("s")
    wid = c * NS + s
    rows = (r0, r1)
    ib = (i0, i1, i2, i3)
    gsem = (g0, g1)
    ssem = (s0, s1)
    isem = (is0, is1, is2, is3)

    zdesc = pltpu.async_copy(zeros_hbm.at[pl.ds(s * RPT, RPT)],
                             zsh.at[pl.ds(s * RPT, RPT)], zsem)
    base = wid * CHUNKS_PW

    pltpu.async_copy(edges_hbm.at[:, pl.ds(base, 1)], ib[0], isem[0])
    pltpu.async_copy(edges_hbm.at[:, pl.ds(base + 1, 1)], ib[1], isem[1])
    zdesc.wait()
    plsc.subcore_barrier()

    def _quad(t, _):
        for kk in range(4):
            j = 4 * t + kk
            kr = kk % 2
            kp = (kk - 1) % 4

            def _wait_rows_free():
                pltpu.make_async_copy(rows[kr], zsh.at[ib[kk].at[0, 0]],
                                      ssem[kr]).wait()
            if kk < 2:
                pl.when(t >= 1)(_wait_rows_free)
            else:
                _wait_rows_free()
            pltpu.make_async_copy(
                edges_hbm.at[:, pl.ds(base + j, 1)], ib[kk],
                isem[kk]).wait()
            pltpu.async_copy(u_hbm.at[ib[kk].at[0, 0]], rows[kr], gsem[kr])

            def _prefetch():
                jn = j + 2
                kn = (kk + 2) % 4
                pltpu.async_copy(edges_hbm.at[:, pl.ds(base + jn, 1)],
                                 ib[kn], isem[kn])
            if kk < 2:
                _prefetch()
            else:
                pl.when(t < CHUNKS_PW // 4 - 1)(_prefetch)

            def _emit_prev_scatter():
                pltpu.make_async_copy(u_hbm.at[ib[kp].at[0, 0]], rows[1 - kr],
                                      gsem[1 - kr]).wait()
                pltpu.async_copy(rows[1 - kr], zsh.at[ib[kp].at[1, 0]],
                                 ssem[1 - kr], add=True)
            if kk == 0:
                pl.when(t >= 1)(_emit_prev_scatter)
            else:
                _emit_prev_scatter()
        return 0
    lax.fori_loop(0, CHUNKS_PW // 4, _quad, 0)

    pltpu.make_async_copy(u_hbm.at[ib[3].at[0, 0]], rows[1], gsem[1]).wait()
    pltpu.async_copy(rows[1], zsh.at[ib[3].at[1, 0]], ssem[1], add=True)
    pltpu.make_async_copy(rows[0], zsh.at[ib[2].at[1, 0]], ssem[0]).wait()
    pltpu.make_async_copy(rows[1], zsh.at[ib[3].at[1, 0]], ssem[1]).wait()
    plsc.subcore_barrier()

    pltpu.sync_copy(zsh.at[pl.ds(s * RPT, RPT)],
                    out_hbm.at[pl.ds(c * N_PAD + s * RPT, RPT)])


@functools.lru_cache(maxsize=None)
def _layer_call():
    return pl.kernel(
        _layer_body,
        out_type=jax.ShapeDtypeStruct((2 * N_PAD, D), jnp.float32),
        mesh=_sc_mesh(),
        scratch_types=(
            [pltpu.VMEM_SHARED((N_PAD, D), jnp.float32)]
            + [pltpu.VMEM((CHUNK, D), jnp.float32)] * 2
            + [pltpu.VMEM((2, 1, CHUNK), jnp.int32)] * 4
            + [pltpu.SemaphoreType.DMA] * 9
        ),
    )


_ROWS_BLK = 512


def _prep_body(dp_ref, x0_ref, u0_ref, scl_ref):
    dp = dp_ref[...]
    degs = dp[0:1, :] + dp[2:3, :]
    degd = dp[1:2, :] + dp[3:4, :]
    dis = jnp.where(degs > 0, lax.rsqrt(jnp.maximum(degs, 1.0)), 0.0)
    did = jnp.where(degd > 0, lax.rsqrt(jnp.maximum(degd, 1.0)), 0.0)
    t = jnp.transpose(jnp.concatenate([dis, did], axis=0), (1, 0))
    disc = t[:, 0:1]
    didc = t[:, 1:2]
    u0_ref[...] = x0_ref[...] * disc
    scl_ref[...] = jnp.concatenate([disc * didc, didc], axis=1)


@functools.lru_cache(maxsize=None)
def _prep_call():
    nblk = N_PAD // _ROWS_BLK
    return pl.pallas_call(
        _prep_body,
        grid=(nblk,),
        in_specs=[
            pl.BlockSpec((4, _ROWS_BLK), lambda i: (0, i)),
            pl.BlockSpec((_ROWS_BLK, D), lambda i: (i, 0)),
        ],
        out_specs=[
            pl.BlockSpec((_ROWS_BLK, D), lambda i: (i, 0)),
            pl.BlockSpec((_ROWS_BLK, 2), lambda i: (i, 0)),
        ],
        out_shape=[
            jax.ShapeDtypeStruct((N_PAD, D), jnp.float32),
            jax.ShapeDtypeStruct((N_PAD, 2), jnp.float32),
        ],
    )


def _merge_body(zp_ref, scl_ref, acc_ref, unext_ref, accout_ref):
    z = zp_ref[0] + zp_ref[1]
    unext_ref[...] = z * scl_ref[:, 0:1]
    accout_ref[...] = acc_ref[...] + z * scl_ref[:, 1:2]


def _final_body(zp_ref, scl_ref, acc_ref, out_ref):
    z = zp_ref[0] + zp_ref[1]
    out_ref[...] = (acc_ref[...] + z * scl_ref[:, 1:2]) * 0.25


@functools.lru_cache(maxsize=None)
def _merge_call():
    nblk = N_PAD // _ROWS_BLK
    return pl.pallas_call(
        _merge_body,
        grid=(nblk,),
        in_specs=[
            pl.BlockSpec((2, _ROWS_BLK, D), lambda i: (0, i, 0)),
            pl.BlockSpec((_ROWS_BLK, 2), lambda i: (i, 0)),
            pl.BlockSpec((_ROWS_BLK, D), lambda i: (i, 0)),
        ],
        out_specs=[
            pl.BlockSpec((_ROWS_BLK, D), lambda i: (i, 0)),
            pl.BlockSpec((_ROWS_BLK, D), lambda i: (i, 0)),
        ],
        out_shape=[
            jax.ShapeDtypeStruct((N_PAD, D), jnp.float32),
            jax.ShapeDtypeStruct((N_PAD, D), jnp.float32),
        ],
    )


@functools.lru_cache(maxsize=None)
def _final_call():
    nblk = N_PAD // _ROWS_BLK
    return pl.pallas_call(
        _final_body,
        grid=(nblk,),
        in_specs=[
            pl.BlockSpec((2, _ROWS_BLK, D), lambda i: (0, i, 0)),
            pl.BlockSpec((_ROWS_BLK, 2), lambda i: (i, 0)),
            pl.BlockSpec((_ROWS_BLK, D), lambda i: (i, 0)),
        ],
        out_specs=pl.BlockSpec((_ROWS_BLK, D), lambda i: (i, 0)),
        out_shape=jax.ShapeDtypeStruct((N_PAD, D), jnp.float32),
    )


def kernel(user_emb, item_emb, edge_index):
    x0 = jnp.concatenate([user_emb, item_emb], axis=0)
    x0p = jnp.pad(x0, ((0, N_PAD - N), (0, 0)))
    pad_idx = (N + (jnp.arange(E_PAD - E, dtype=jnp.int32) % (N_PAD - N))
               ).astype(jnp.int32)
    pad2 = jnp.stack([pad_idx, pad_idx])
    edges = jnp.concatenate([edge_index, pad2], axis=1)
    edges4 = edges.reshape(2, NW, CHUNKS_PW, CHUNK)
    edges3 = edges.reshape(2, NW * CHUNKS_PW, CHUNK)
    zeros = jnp.zeros((N_PAD, D), jnp.float32)

    degpart = _deg_call()(edges4)
    dp = degpart.reshape(4, N_PAD)
    u, scl = _prep_call()(dp, x0p)

    acc = x0p
    out = None
    for l in range(N_LAYERS):
        zp = _layer_call()(u, edges3, zeros).reshape(2, N_PAD, D)
        if l < N_LAYERS - 1:
            u, acc = _merge_call()(zp, scl, acc)
        else:
            out = _final_call()(zp, scl, acc)
    return out[:N_USERS], out[N_USERS:N]


# speedup vs baseline: 20.3168x; 1.0060x over previous
import functools

import jax
import jax.numpy as jnp
from jax import lax
from jax.experimental import pallas as pl
from jax.experimental.pallas import tpu as pltpu
from jax.experimental.pallas import tpu_sc as plsc

N_USERS = 4000
N_ITEMS = 6000
N = N_USERS + N_ITEMS
D = 128
E = 320000
N_LAYERS = 3

NC = 2
NS = 16
NW = NC * NS
CHUNK = 128
CHUNKS_PW = 80
E_PAD = NW * CHUNK * CHUNKS_PW
N_PAD = 10240
RPT = N_PAD // NS
NBUF = 2


def _sc_mesh():
    return plsc.VectorSubcoreMesh(core_axis_name="c", subcore_axis_name="s")


def _deg_body(edges_hbm, out_hbm, dsrc_sh, ddst_sh, ibuf, ones_v,
              sa0, sa1, sa2, sa3, sb0, sb1, sb2, sb3, isem):
    c = lax.axis_index("c")
    s = lax.axis_index("s")
    wid = c * NS + s
    sa = (sa0, sa1, sa2, sa3)
    sb = (sb0, sb1, sb2, sb3)

    idesc = pltpu.async_copy(edges_hbm.at[:, wid], ibuf, isem)

    def _zfill(i, _):
        ones_v[pl.ds(i * 16, 16)] = jnp.zeros((16,), jnp.float32)
        return 0
    lax.fori_loop(0, CHUNK // 16, _zfill, 0)
    for off in range(0, RPT, CHUNK):
        w = min(CHUNK, RPT - off)
        pltpu.sync_copy(ones_v.at[pl.ds(0, w)],
                        dsrc_sh.at[pl.ds(s * RPT + off, w)])
        pltpu.sync_copy(ones_v.at[pl.ds(0, w)],
                        ddst_sh.at[pl.ds(s * RPT + off, w)])

    def _fill(i, _):
        ones_v[pl.ds(i * 16, 16)] = jnp.ones((16,), jnp.float32)
        return 0
    lax.fori_loop(0, CHUNK // 16, _fill, 0)
    idesc.wait()
    plsc.subcore_barrier()

    def _quad(t, _):
        for kk in range(4):
            j = 4 * t + kk

            def _wait_prev():
                pltpu.make_async_copy(ones_v, dsrc_sh.at[ibuf.at[0, j]],
                                      sa[kk]).wait()
                pltpu.make_async_copy(ones_v, ddst_sh.at[ibuf.at[1, j]],
                                      sb[kk]).wait()
            pl.when(t >= 1)(_wait_prev)
            pltpu.async_copy(ones_v, dsrc_sh.at[ibuf.at[0, j]], sa[kk],
                             add=True)
            pltpu.async_copy(ones_v, ddst_sh.at[ibuf.at[1, j]], sb[kk],
                             add=True)
        return 0
    lax.fori_loop(0, CHUNKS_PW // 4, _quad, 0)
    for kk in range(4):
        j = CHUNKS_PW - 4 + kk
        pltpu.make_async_copy(ones_v, dsrc_sh.at[ibuf.at[0, j]], sa[kk]).wait()
        pltpu.make_async_copy(ones_v, ddst_sh.at[ibuf.at[1, j]], sb[kk]).wait()
    plsc.subcore_barrier()

    pltpu.sync_copy(dsrc_sh.at[pl.ds(s * RPT, RPT)],
                    out_hbm.at[pl.ds((c * 2 + 0) * N_PAD + s * RPT, RPT)])
    pltpu.sync_copy(ddst_sh.at[pl.ds(s * RPT, RPT)],
                    out_hbm.at[pl.ds((c * 2 + 1) * N_PAD + s * RPT, RPT)])


@functools.lru_cache(maxsize=None)
def _deg_call():
    return pl.kernel(
        _deg_body,
        out_type=jax.ShapeDtypeStruct((4 * N_PAD,), jnp.float32),
        mesh=_sc_mesh(),
        scratch_types=[
            pltpu.VMEM_SHARED((N_PAD,), jnp.float32),
            pltpu.VMEM_SHARED((N_PAD,), jnp.float32),
            pltpu.VMEM((2, CHUNKS_PW, CHUNK), jnp.int32),
            pltpu.VMEM((CHUNK,), jnp.float32),
        ] + [pltpu.SemaphoreType.DMA] * 9,
    )


def _layer_body(u_hbm, edges_hbm, zeros_hbm, out_hbm, zsh,
                r0, r1, i0, i1, i2, i3,
                g0, g1, s0, s1, is0, is1, is2, is3, zsem):
    c = lax.axis_index("c")
    s = lax.axis_index("s")
    wid = c * NS + s
    rows = (r0, r1)
    ib = (i0, i1, i2, i3)
    gsem = (g0, g1)
    ssem = (s0, s1)
    isem = (is0, is1, is2, is3)

    zdesc = pltpu.async_copy(zeros_hbm.at[pl.ds(s * RPT, RPT)],
                             zsh.at[pl.ds(s * RPT, RPT)], zsem)
    base = wid * CHUNKS_PW

    pltpu.async_copy(edges_hbm.at[:, pl.ds(base, 1)], ib[0], isem[0])
    pltpu.async_copy(edges_hbm.at[:, pl.ds(base + 1, 1)], ib[1], isem[1])
    zdesc.wait()
    plsc.subcore_barrier()

    def _quad(t, _):
        for kk in range(4):
            j = 4 * t + kk
            kr = kk % 2
            kp = (kk - 1) % 4

            def _wait_rows_free():
                pltpu.make_async_copy(rows[kr], zsh.at[ib[kk].at[0, 0]],
                                      ssem[kr]).wait()
            if kk < 2:
                pl.when(t >= 1)(_wait_rows_free)
            else:
                _wait_rows_free()
            pltpu.make_async_copy(
                edges_hbm.at[:, pl.ds(base + j, 1)], ib[kk],
                isem[kk]).wait()
            pltpu.async_copy(u_hbm.at[ib[kk].at[0, 0]], rows[kr], gsem[kr])

            def _prefetch():
                jn = j + 2
                kn = (kk + 2) % 4
                pltpu.async_copy(edges_hbm.at[:, pl.ds(base + jn, 1)],
                                 ib[kn], isem[kn])
            if kk < 2:
                _prefetch()
            else:
                pl.when(t < CHUNKS_PW // 4 - 1)(_prefetch)

            def _emit_prev_scatter():
                pltpu.make_async_copy(u_hbm.at[ib[kp].at[0, 0]], rows[1 - kr],
                                      gsem[1 - kr]).wait()
                pltpu.async_copy(rows[1 - kr], zsh.at[ib[kp].at[1, 0]],
                                 ssem[1 - kr], add=True)
            if kk == 0:
                pl.when(t >= 1)(_emit_prev_scatter)
            else:
                _emit_prev_scatter()
        return 0
    lax.fori_loop(0, CHUNKS_PW // 4, _quad, 0)

    pltpu.make_async_copy(u_hbm.at[ib[3].at[0, 0]], rows[1], gsem[1]).wait()
    pltpu.async_copy(rows[1], zsh.at[ib[3].at[1, 0]], ssem[1], add=True)
    pltpu.make_async_copy(rows[0], zsh.at[ib[2].at[1, 0]], ssem[0]).wait()
    pltpu.make_async_copy(rows[1], zsh.at[ib[3].at[1, 0]], ssem[1]).wait()
    plsc.subcore_barrier()

    pltpu.sync_copy(zsh.at[pl.ds(s * RPT, RPT)],
                    out_hbm.at[pl.ds(c * N_PAD + s * RPT, RPT)])


@functools.lru_cache(maxsize=None)
def _layer_call():
    return pl.kernel(
        _layer_body,
        out_type=jax.ShapeDtypeStruct((2 * N_PAD, D), jnp.float32),
        mesh=_sc_mesh(),
        scratch_types=(
            [pltpu.VMEM_SHARED((N_PAD, D), jnp.float32)]
            + [pltpu.VMEM((CHUNK, D), jnp.float32)] * 2
            + [pltpu.VMEM((2, 1, CHUNK), jnp.int32)] * 4
            + [pltpu.SemaphoreType.DMA] * 9
        ),
    )


_ROWS_BLK = 512


def _prep_body(dp_ref, x0_ref, u0_ref, scl_ref):
    dp = dp_ref[...]
    degs = dp[0:1, :] + dp[2:3, :]
    degd = dp[1:2, :] + dp[3:4, :]
    dis = jnp.where(degs > 0, lax.rsqrt(jnp.maximum(degs, 1.0)), 0.0)
    did = jnp.where(degd > 0, lax.rsqrt(jnp.maximum(degd, 1.0)), 0.0)
    t = jnp.transpose(jnp.concatenate([dis, did], axis=0), (1, 0))
    disc = t[:, 0:1]
    didc = t[:, 1:2]
    u0_ref[...] = x0_ref[...] * disc
    scl_ref[...] = jnp.concatenate([disc * didc, didc], axis=1)


@functools.lru_cache(maxsize=None)
def _prep_call():
    nblk = N_PAD // _ROWS_BLK
    return pl.pallas_call(
        _prep_body,
        grid=(nblk,),
        in_specs=[
            pl.BlockSpec((4, _ROWS_BLK), lambda i: (0, i)),
            pl.BlockSpec((_ROWS_BLK, D), lambda i: (i, 0)),
        ],
        out_specs=[
            pl.BlockSpec((_ROWS_BLK, D), lambda i: (i, 0)),
            pl.BlockSpec((_ROWS_BLK, 2), lambda i: (i, 0)),
        ],
        out_shape=[
            jax.ShapeDtypeStruct((N_PAD, D), jnp.float32),
            jax.ShapeDtypeStruct((N_PAD, 2), jnp.float32),
        ],
    )


def _mergeu_body(zp_ref, scl_ref, unext_ref):
    unext_ref[...] = (zp_ref[0] + zp_ref[1]) * scl_ref[:, 0:1]


def _acc_body(zp_ref, scl_ref, acc_ref, accout_ref):
    accout_ref[...] = acc_ref[...] + (zp_ref[0] + zp_ref[1]) * scl_ref[:, 1:2]


def _final_body(zp_ref, scl_ref, acc_ref, out_ref):
    z = zp_ref[0] + zp_ref[1]
    out_ref[...] = (acc_ref[...] + z * scl_ref[:, 1:2]) * 0.25


@functools.lru_cache(maxsize=None)
def _mergeu_call():
    nblk = N_PAD // _ROWS_BLK
    return pl.pallas_call(
        _mergeu_body,
        grid=(nblk,),
        in_specs=[
            pl.BlockSpec((2, _ROWS_BLK, D), lambda i: (0, i, 0)),
            pl.BlockSpec((_ROWS_BLK, 2), lambda i: (i, 0)),
        ],
        out_specs=pl.BlockSpec((_ROWS_BLK, D), lambda i: (i, 0)),
        out_shape=jax.ShapeDtypeStruct((N_PAD, D), jnp.float32),
    )


@functools.lru_cache(maxsize=None)
def _acc_call():
    nblk = N_PAD // _ROWS_BLK
    return pl.pallas_call(
        _acc_body,
        grid=(nblk,),
        in_specs=[
            pl.BlockSpec((2, _ROWS_BLK, D), lambda i: (0, i, 0)),
            pl.BlockSpec((_ROWS_BLK, 2), lambda i: (i, 0)),
            pl.BlockSpec((_ROWS_BLK, D), lambda i: (i, 0)),
        ],
        out_specs=pl.BlockSpec((_ROWS_BLK, D), lambda i: (i, 0)),
        out_shape=jax.ShapeDtypeStruct((N_PAD, D), jnp.float32),
    )


@functools.lru_cache(maxsize=None)
def _final_call():
    nblk = N_PAD // _ROWS_BLK
    return pl.pallas_call(
        _final_body,
        grid=(nblk,),
        in_specs=[
            pl.BlockSpec((2, _ROWS_BLK, D), lambda i: (0, i, 0)),
            pl.BlockSpec((_ROWS_BLK, 2), lambda i: (i, 0)),
            pl.BlockSpec((_ROWS_BLK, D), lambda i: (i, 0)),
        ],
        out_specs=pl.BlockSpec((_ROWS_BLK, D), lambda i: (i, 0)),
        out_shape=jax.ShapeDtypeStruct((N_PAD, D), jnp.float32),
    )


def kernel(user_emb, item_emb, edge_index):
    x0 = jnp.concatenate([user_emb, item_emb], axis=0)
    x0p = jnp.pad(x0, ((0, N_PAD - N), (0, 0)))
    pad_idx = (N + (jnp.arange(E_PAD - E, dtype=jnp.int32) % (N_PAD - N))
               ).astype(jnp.int32)
    pad2 = jnp.stack([pad_idx, pad_idx])
    edges = jnp.concatenate([edge_index, pad2], axis=1)
    edges4 = edges.reshape(2, NW, CHUNKS_PW, CHUNK)
    edges3 = edges.reshape(2, NW * CHUNKS_PW, CHUNK)
    zeros = jnp.zeros((N_PAD, D), jnp.float32)

    degpart = _deg_call()(edges4)
    dp = degpart.reshape(4, N_PAD)
    u, scl = _prep_call()(dp, x0p)

    acc = x0p
    out = None
    for l in range(N_LAYERS):
        zp = _layer_call()(u, edges3, zeros).reshape(2, N_PAD, D)
        if l < N_LAYERS - 1:
            u = _mergeu_call()(zp, scl)
            acc = _acc_call()(zp, scl, acc)
        else:
            out = _final_call()(zp, scl, acc)
    return out[:N_USERS], out[N_USERS:N]


# speedup vs baseline: 20.4572x; 1.0069x over previous
import functools

import jax
import jax.numpy as jnp
from jax import lax
from jax.experimental import pallas as pl
from jax.experimental.pallas import tpu as pltpu
from jax.experimental.pallas import tpu_sc as plsc

N_USERS = 4000
N_ITEMS = 6000
N = N_USERS + N_ITEMS
D = 128
E = 320000
N_LAYERS = 3

NC = 2
NS = 16
NW = NC * NS
CHUNK = 128
CHUNKS_PW = 80
E_PAD = NW * CHUNK * CHUNKS_PW
N_PAD = 10240
RPT = N_PAD // NS
NBUF = 2


def _sc_mesh():
    return plsc.VectorSubcoreMesh(core_axis_name="c", subcore_axis_name="s")


def _deg_body(edges_hbm, out_hbm, dsrc_sh, ddst_sh, ibuf, ones_v,
              sa0, sa1, sa2, sa3, sb0, sb1, sb2, sb3, isem):
    c = lax.axis_index("c")
    s = lax.axis_index("s")
    wid = c * NS + s
    sa = (sa0, sa1, sa2, sa3)
    sb = (sb0, sb1, sb2, sb3)

    idesc = pltpu.async_copy(edges_hbm.at[:, wid], ibuf, isem)

    def _zfill(i, _):
        ones_v[pl.ds(i * 16, 16)] = jnp.zeros((16,), jnp.float32)
        return 0
    lax.fori_loop(0, CHUNK // 16, _zfill, 0)
    for off in range(0, RPT, CHUNK):
        w = min(CHUNK, RPT - off)
        pltpu.sync_copy(ones_v.at[pl.ds(0, w)],
                        dsrc_sh.at[pl.ds(s * RPT + off, w)])
        pltpu.sync_copy(ones_v.at[pl.ds(0, w)],
                        ddst_sh.at[pl.ds(s * RPT + off, w)])

    def _fill(i, _):
        ones_v[pl.ds(i * 16, 16)] = jnp.ones((16,), jnp.float32)
        return 0
    lax.fori_loop(0, CHUNK // 16, _fill, 0)
    idesc.wait()
    plsc.subcore_barrier()

    def _quad(t, _):
        for kk in range(4):
            j = 4 * t + kk

            def _wait_prev():
                pltpu.make_async_copy(ones_v, dsrc_sh.at[ibuf.at[0, j]],
                                      sa[kk]).wait()
                pltpu.make_async_copy(ones_v, ddst_sh.at[ibuf.at[1, j]],
                                      sb[kk]).wait()
            pl.when(t >= 1)(_wait_prev)
            pltpu.async_copy(ones_v, dsrc_sh.at[ibuf.at[0, j]], sa[kk],
                             add=True)
            pltpu.async_copy(ones_v, ddst_sh.at[ibuf.at[1, j]], sb[kk],
                             add=True)
        return 0
    lax.fori_loop(0, CHUNKS_PW // 4, _quad, 0)
    for kk in range(4):
        j = CHUNKS_PW - 4 + kk
        pltpu.make_async_copy(ones_v, dsrc_sh.at[ibuf.at[0, j]], sa[kk]).wait()
        pltpu.make_async_copy(ones_v, ddst_sh.at[ibuf.at[1, j]], sb[kk]).wait()
    plsc.subcore_barrier()

    pltpu.sync_copy(dsrc_sh.at[pl.ds(s * RPT, RPT)],
                    out_hbm.at[pl.ds((c * 2 + 0) * N_PAD + s * RPT, RPT)])
    pltpu.sync_copy(ddst_sh.at[pl.ds(s * RPT, RPT)],
                    out_hbm.at[pl.ds((c * 2 + 1) * N_PAD + s * RPT, RPT)])


@functools.lru_cache(maxsize=None)
def _deg_call():
    return pl.kernel(
        _deg_body,
        out_type=jax.ShapeDtypeStruct((4 * N_PAD,), jnp.float32),
        mesh=_sc_mesh(),
        scratch_types=[
            pltpu.VMEM_SHARED((N_PAD,), jnp.float32),
            pltpu.VMEM_SHARED((N_PAD,), jnp.float32),
            pltpu.VMEM((2, CHUNKS_PW, CHUNK), jnp.int32),
            pltpu.VMEM((CHUNK,), jnp.float32),
        ] + [pltpu.SemaphoreType.DMA] * 9,
    )


def _layer_body(u_hbm, edges_hbm, zeros_hbm, out_hbm, zsh,
                r0, r1, i0, i1, i2, i3,
                g0, g1, s0, s1, is0, is1, is2, is3, zsem):
    c = lax.axis_index("c")
    s = lax.axis_index("s")
    wid = c * NS + s
    rows = (r0, r1)
    ib = (i0, i1, i2, i3)
    gsem = (g0, g1)
    ssem = (s0, s1)
    isem = (is0, is1, is2, is3)

    zdesc = pltpu.async_copy(zeros_hbm.at[pl.ds(s * RPT, RPT)],
                             zsh.at[pl.ds(s * RPT, RPT)], zsem)
    base = wid * CHUNKS_PW

    pltpu.async_copy(edges_hbm.at[:, pl.ds(base, 1)], ib[0], isem[0])
    pltpu.async_copy(edges_hbm.at[:, pl.ds(base + 1, 1)], ib[1], isem[1])
    zdesc.wait()
    plsc.subcore_barrier()

    def _quad(t, _):
        for kk in range(4):
            j = 4 * t + kk
            kr = kk % 2
            kp = (kk - 1) % 4

            def _wait_rows_free():
                pltpu.make_async_copy(rows[kr], zsh.at[ib[kk].at[0, 0]],
                                      ssem[kr]).wait()
            if kk < 2:
                pl.when(t >= 1)(_wait_rows_free)
            else:
                _wait_rows_free()
            pltpu.make_async_copy(
                edges_hbm.at[:, pl.ds(base + j, 1)], ib[kk],
                isem[kk]).wait()
            pltpu.async_copy(u_hbm.at[ib[kk].at[0, 0]], rows[kr], gsem[kr])

            def _prefetch():
                jn = j + 2
                kn = (kk + 2) % 4
                pltpu.async_copy(edges_hbm.at[:, pl.ds(base + jn, 1)],
                                 ib[kn], isem[kn])
            if kk < 2:
                _prefetch()
            else:
                pl.when(t < CHUNKS_PW // 4 - 1)(_prefetch)

            def _emit_prev_scatter():
                pltpu.make_async_copy(u_hbm.at[ib[kp].at[0, 0]], rows[1 - kr],
                                      gsem[1 - kr]).wait()
                pltpu.async_copy(rows[1 - kr], zsh.at[ib[kp].at[1, 0]],
                                 ssem[1 - kr], add=True)
            if kk == 0:
                pl.when(t >= 1)(_emit_prev_scatter)
            else:
                _emit_prev_scatter()
        return 0
    lax.fori_loop(0, CHUNKS_PW // 4, _quad, 0)

    pltpu.make_async_copy(u_hbm.at[ib[3].at[0, 0]], rows[1], gsem[1]).wait()
    pltpu.async_copy(rows[1], zsh.at[ib[3].at[1, 0]], ssem[1], add=True)
    pltpu.make_async_copy(rows[0], zsh.at[ib[2].at[1, 0]], ssem[0]).wait()
    pltpu.make_async_copy(rows[1], zsh.at[ib[3].at[1, 0]], ssem[1]).wait()
    plsc.subcore_barrier()

    pltpu.sync_copy(zsh.at[pl.ds(s * RPT, RPT)],
                    out_hbm.at[pl.ds(c * N_PAD + s * RPT, RPT)])


@functools.lru_cache(maxsize=None)
def _layer_call():
    return pl.kernel(
        _layer_body,
        out_type=jax.ShapeDtypeStruct((2 * N_PAD, D), jnp.float32),
        mesh=_sc_mesh(),
        scratch_types=(
            [pltpu.VMEM_SHARED((N_PAD, D), jnp.float32)]
            + [pltpu.VMEM((CHUNK, D), jnp.float32)] * 2
            + [pltpu.VMEM((2, 1, CHUNK), jnp.int32)] * 4
            + [pltpu.SemaphoreType.DMA] * 9
        ),
    )


_ROWS_BLK = 512


def _prep_body(dp_ref, x0_ref, u0_ref, scl_ref):
    dp = dp_ref[...]
    degs = dp[0:1, :] + dp[2:3, :]
    degd = dp[1:2, :] + dp[3:4, :]
    dis = jnp.where(degs > 0, lax.rsqrt(jnp.maximum(degs, 1.0)), 0.0)
    did = jnp.where(degd > 0, lax.rsqrt(jnp.maximum(degd, 1.0)), 0.0)
    t = jnp.transpose(jnp.concatenate([dis, did], axis=0), (1, 0))
    disc = t[:, 0:1]
    didc = t[:, 1:2]
    u0_ref[...] = x0_ref[...] * disc
    scl_ref[...] = jnp.concatenate([disc * didc, didc], axis=1)


@functools.lru_cache(maxsize=None)
def _prep_call():
    nblk = N_PAD // _ROWS_BLK
    return pl.pallas_call(
        _prep_body,
        grid=(nblk,),
        in_specs=[
            pl.BlockSpec((4, _ROWS_BLK), lambda i: (0, i)),
            pl.BlockSpec((_ROWS_BLK, D), lambda i: (i, 0)),
        ],
        out_specs=[
            pl.BlockSpec((_ROWS_BLK, D), lambda i: (i, 0)),
            pl.BlockSpec((_ROWS_BLK, 2), lambda i: (i, 0)),
        ],
        out_shape=[
            jax.ShapeDtypeStruct((N_PAD, D), jnp.float32),
            jax.ShapeDtypeStruct((N_PAD, 2), jnp.float32),
        ],
    )


def _mergeu_body(zp_ref, scl_ref, unext_ref):
    unext_ref[...] = (zp_ref[0] + zp_ref[1]) * scl_ref[:, 0:1]


def _acc_body(zp_ref, scl_ref, acc_ref, accout_ref):
    accout_ref[...] = acc_ref[...] + (zp_ref[0] + zp_ref[1]) * scl_ref[:, 1:2]


def _final_body(zp_ref, scl_ref, acc_ref, out_ref):
    z = zp_ref[0] + zp_ref[1]
    out_ref[...] = (acc_ref[...] + z * scl_ref[:, 1:2]) * 0.25


_FBLK = 400


@functools.lru_cache(maxsize=None)
def _final_slice_call(nrows, row0):
    nblk = nrows // _FBLK
    off = row0 // _FBLK
    return pl.pallas_call(
        _final_body,
        grid=(nblk,),
        in_specs=[
            pl.BlockSpec((2, _FBLK, D), lambda i: (0, i + off, 0)),
            pl.BlockSpec((_FBLK, 2), lambda i: (i + off, 0)),
            pl.BlockSpec((_FBLK, D), lambda i: (i + off, 0)),
        ],
        out_specs=pl.BlockSpec((_FBLK, D), lambda i: (i, 0)),
        out_shape=jax.ShapeDtypeStruct((nrows, D), jnp.float32),
    )


@functools.lru_cache(maxsize=None)
def _mergeu_call():
    nblk = N_PAD // _ROWS_BLK
    return pl.pallas_call(
        _mergeu_body,
        grid=(nblk,),
        in_specs=[
            pl.BlockSpec((2, _ROWS_BLK, D), lambda i: (0, i, 0)),
            pl.BlockSpec((_ROWS_BLK, 2), lambda i: (i, 0)),
        ],
        out_specs=pl.BlockSpec((_ROWS_BLK, D), lambda i: (i, 0)),
        out_shape=jax.ShapeDtypeStruct((N_PAD, D), jnp.float32),
    )


@functools.lru_cache(maxsize=None)
def _acc_call():
    nblk = N_PAD // _ROWS_BLK
    return pl.pallas_call(
        _acc_body,
        grid=(nblk,),
        in_specs=[
            pl.BlockSpec((2, _ROWS_BLK, D), lambda i: (0, i, 0)),
            pl.BlockSpec((_ROWS_BLK, 2), lambda i: (i, 0)),
            pl.BlockSpec((_ROWS_BLK, D), lambda i: (i, 0)),
        ],
        out_specs=pl.BlockSpec((_ROWS_BLK, D), lambda i: (i, 0)),
        out_shape=jax.ShapeDtypeStruct((N_PAD, D), jnp.float32),
    )


def kernel(user_emb, item_emb, edge_index):
    x0 = jnp.concatenate([user_emb, item_emb], axis=0)
    x0p = jnp.pad(x0, ((0, N_PAD - N), (0, 0)))
    pad_idx = (N + (jnp.arange(E_PAD - E, dtype=jnp.int32) % (N_PAD - N))
               ).astype(jnp.int32)
    pad2 = jnp.stack([pad_idx, pad_idx])
    edges = jnp.concatenate([edge_index, pad2], axis=1)
    edges4 = edges.reshape(2, NW, CHUNKS_PW, CHUNK)
    edges3 = edges.reshape(2, NW * CHUNKS_PW, CHUNK)
    zeros = jnp.zeros((N_PAD, D), jnp.float32)

    degpart = _deg_call()(edges4)
    dp = degpart.reshape(4, N_PAD)
    u, scl = _prep_call()(dp, x0p)

    acc = x0p
    out = None
    for l in range(N_LAYERS):
        zp = _layer_call()(u, edges3, zeros).reshape(2, N_PAD, D)
        if l < N_LAYERS - 1:
            u = _mergeu_call()(zp, scl)
            acc = _acc_call()(zp, scl, acc)
        else:
            users = _final_slice_call(N_USERS, 0)(zp, scl, acc)
            items = _final_slice_call(N_ITEMS, N_USERS)(zp, scl, acc)
    return users, items
